# baseline (speedup 1.0000x reference)
import jax
import jax.numpy as jnp
from jax.experimental import pallas as pl
from jax.experimental.pallas import tpu as pltpu


def _ceil_to(n, m):
    return ((n + m - 1) // m) * m


def _make_body(Tc, HB, n_split):
    def body(x_ref, h0_ref, wt_ref, wh_ref, o_ref, h_ref):
        tc = pl.program_id(1)

        @pl.when(tc == 0)
        def _():
            h_ref[...] = h0_ref[...]

        wh = wh_ref[...]
        wt = wt_ref[...]
        hs = [h_ref[i * HB:(i + 1) * HB] for i in range(n_split)]
        for s in range(Tc):
            xs = x_ref[s].astype(jnp.bfloat16)
            pre = jnp.dot(xs, wt, preferred_element_type=jnp.float32)
            for i in range(n_split):
                rec = jnp.dot(hs[i], wh, preferred_element_type=jnp.float32)
                y = jnp.tanh(pre[i * HB:(i + 1) * HB] + rec)
                o_ref[s, i * HB:(i + 1) * HB] = y
                hs[i] = y.astype(jnp.bfloat16)
        for i in range(n_split):
            h_ref[i * HB:(i + 1) * HB] = hs[i]

    return body


def kernel(x_seq, h0, W, W_hidden):
    T, B, in_dim = x_seq.shape
    out_dim = W.shape[0]
    dtype = x_seq.dtype

    Dp = _ceil_to(in_dim, 128)
    Np = _ceil_to(out_dim, 128)
    TB = _ceil_to(B, 8)
    Bp = TB
    Tc = 8 if T % 8 == 0 else 1
    Tp = _ceil_to(T, Tc)
    n_split = 2 if TB % 16 == 0 else 1
    HB = TB // n_split

    wt = jnp.zeros((Dp, Np), jnp.bfloat16).at[:in_dim, :out_dim].set(
        W.T.astype(jnp.bfloat16))
    wh = jnp.zeros((Dp, Np), jnp.bfloat16).at[:in_dim, :out_dim].set(
        W_hidden.T.astype(jnp.bfloat16))
    h0_p = jnp.zeros((Bp, Dp), jnp.bfloat16).at[:B, :in_dim].set(
        h0.astype(jnp.bfloat16))
    if (Tp, Bp, Dp) != (T, B, in_dim):
        x_p = jnp.zeros((Tp, Bp, Dp), dtype).at[:T, :B, :in_dim].set(x_seq)
    else:
        x_p = x_seq

    cost = pl.CostEstimate(
        flops=2 * 2 * Tp * Bp * Dp * Np,
        transcendentals=Tp * Bp * Np,
        bytes_accessed=4 * (2 * Tp * Bp * Np),
    )

    out_p = pl.pallas_call(
        _make_body(Tc, HB, n_split),
        out_shape=jax.ShapeDtypeStruct((Tp, Bp, Np), jnp.float32),
        grid=(Bp // TB, Tp // Tc),
        in_specs=[
            pl.BlockSpec((Tc, TB, Dp), lambda b, t: (t, b, 0)),
            pl.BlockSpec((TB, Dp), lambda b, t: (b, 0)),
            pl.BlockSpec((Dp, Np), lambda b, t: (0, 0)),
            pl.BlockSpec((Dp, Np), lambda b, t: (0, 0)),
        ],
        out_specs=pl.BlockSpec((Tc, TB, Np), lambda b, t: (t, b, 0)),
        scratch_shapes=[pltpu.VMEM((TB, Dp), jnp.bfloat16)],
        compiler_params=pltpu.CompilerParams(
            dimension_semantics=("parallel", "arbitrary"),
        ),
        cost_estimate=cost,
    )(x_p, h0_p, wt, wh)

    if (Tp, Bp, Np) != (T, B, out_dim):
        out_p = out_p[:T, :B, :out_dim]
    return out_p.astype(dtype)


# speedup vs baseline: 1.1162x; 1.1162x over previous
import jax
import jax.numpy as jnp
from jax.experimental import pallas as pl
from jax.experimental.pallas import tpu as pltpu


def _ceil_to(n, m):
    return ((n + m - 1) // m) * m


def _chunk_schedule(T):
    if T >= 64 and T % 16 == 0:
        return [8] + [16] * ((T - 16) // 16) + [8]
    if T % 8 == 0:
        return [8] * (T // 8)
    return [T]


def _make_body(cs, HB, n_split):
    NC = len(cs)
    offs = [sum(cs[:i]) for i in range(NC)]

    def body(x_hbm, h0_ref, wt_ref, wh_ref, o_hbm, xbuf, ybuf, in_sem,
             out_sem):
        def in_cp(c):
            return pltpu.make_async_copy(
                x_hbm.at[pl.ds(offs[c], cs[c])],
                xbuf.at[c % 2, pl.ds(0, cs[c])], in_sem.at[c % 2])

        def out_cp(c):
            return pltpu.make_async_copy(
                ybuf.at[c % 2, pl.ds(0, cs[c])],
                o_hbm.at[pl.ds(offs[c], cs[c])], out_sem.at[c % 2])

        in_cp(0).start()
        wt = wt_ref[...]
        wh = wh_ref[...]
        hs = [h0_ref[i * HB:(i + 1) * HB] for i in range(n_split)]
        for c in range(NC):
            if c + 1 < NC:
                in_cp(c + 1).start()
            in_cp(c).wait()
            if c >= 2:
                out_cp(c - 2).wait()
            for s in range(cs[c]):
                xs = xbuf[c % 2, s].astype(jnp.bfloat16)
                pre = jnp.dot(xs, wt, preferred_element_type=jnp.float32)
                for i in range(n_split):
                    rec = jnp.dot(hs[i], wh,
                                  preferred_element_type=jnp.float32)
                    y = jnp.tanh(pre[i * HB:(i + 1) * HB] + rec)
                    ybuf[c % 2, s, i * HB:(i + 1) * HB] = y
                    hs[i] = y.astype(jnp.bfloat16)
            out_cp(c).start()
        if NC >= 2:
            out_cp(NC - 2).wait()
        out_cp(NC - 1).wait()

    return body


def kernel(x_seq, h0, W, W_hidden):
    T, B, in_dim = x_seq.shape
    out_dim = W.shape[0]
    dtype = x_seq.dtype

    Dp = _ceil_to(in_dim, 128)
    Np = _ceil_to(out_dim, 128)
    TB = _ceil_to(B, 8)
    cs = _chunk_schedule(T)
    Cmax = max(cs)
    n_split = 2 if TB % 16 == 0 else 1
    HB = TB // n_split

    wt = jnp.zeros((Dp, Np), jnp.bfloat16).at[:in_dim, :out_dim].set(
        W.T.astype(jnp.bfloat16))
    wh = jnp.zeros((Dp, Np), jnp.bfloat16).at[:in_dim, :out_dim].set(
        W_hidden.T.astype(jnp.bfloat16))
    h0_p = jnp.zeros((TB, Dp), jnp.bfloat16).at[:B, :in_dim].set(
        h0.astype(jnp.bfloat16))
    if (B, in_dim) != (TB, Dp):
        x_p = jnp.zeros((T, TB, Dp), dtype).at[:, :B, :in_dim].set(x_seq)
    else:
        x_p = x_seq

    cost = pl.CostEstimate(
        flops=2 * 2 * T * TB * Dp * Np,
        transcendentals=T * TB * Np,
        bytes_accessed=4 * (2 * T * TB * Np),
    )

    out_p = pl.pallas_call(
        _make_body(cs, HB, n_split),
        out_shape=jax.ShapeDtypeStruct((T, TB, Np), jnp.float32),
        in_specs=[
            pl.BlockSpec(memory_space=pltpu.HBM),
            pl.BlockSpec(memory_space=pltpu.VMEM),
            pl.BlockSpec(memory_space=pltpu.VMEM),
            pl.BlockSpec(memory_space=pltpu.VMEM),
        ],
        out_specs=pl.BlockSpec(memory_space=pltpu.HBM),
        scratch_shapes=[
            pltpu.VMEM((2, Cmax, TB, Dp), jnp.float32),
            pltpu.VMEM((2, Cmax, TB, Np), jnp.float32),
            pltpu.SemaphoreType.DMA((2,)),
            pltpu.SemaphoreType.DMA((2,)),
        ],
        cost_estimate=cost,
    )(x_p, h0_p, wt, wh)

    if (TB, Np) != (B, out_dim):
        out_p = out_p[:, :B, :out_dim]
    return out_p.astype(dtype)


# speedup vs baseline: 1.1602x; 1.0394x over previous
import jax
import jax.numpy as jnp
from jax.experimental import pallas as pl
from jax.experimental.pallas import tpu as pltpu


def _ceil_to(n, m):
    return ((n + m - 1) // m) * m


def _chunk_schedule(T):
    if T >= 64 and T % 16 == 0:
        return [4, 12] + [16] * ((T - 32) // 16) + [12, 4]
    if T % 8 == 0:
        return [8] * (T // 8)
    return [T]


def _make_body(cs, HB, n_split):
    NC = len(cs)
    offs = [sum(cs[:i]) for i in range(NC)]

    def body(x_hbm, h0_ref, wt_ref, wh_ref, o_hbm, xbuf, ybuf, in_sem,
             out_sem):
        def in_cp(c):
            return pltpu.make_async_copy(
                x_hbm.at[pl.ds(offs[c], cs[c])],
                xbuf.at[c % 3, pl.ds(0, cs[c])], in_sem.at[c % 3])

        def out_cp(c):
            return pltpu.make_async_copy(
                ybuf.at[c % 2, pl.ds(0, cs[c])],
                o_hbm.at[pl.ds(offs[c], cs[c])], out_sem.at[c % 2])

        in_cp(0).start()
        if NC >= 2:
            in_cp(1).start()
        wt = wt_ref[...]
        wh = wh_ref[...]
        hs = [h0_ref[i * HB:(i + 1) * HB] for i in range(n_split)]
        for c in range(NC):
            if c + 2 < NC:
                in_cp(c + 2).start()
            in_cp(c).wait()
            if c >= 2:
                out_cp(c - 2).wait()
            for s in range(cs[c]):
                xs = xbuf[c % 3, s].astype(jnp.bfloat16)
                pre = jnp.dot(xs, wt, preferred_element_type=jnp.float32)
                for i in range(n_split):
                    rec = jnp.dot(hs[i], wh,
                                  preferred_element_type=jnp.float32)
                    y = jnp.tanh(pre[i * HB:(i + 1) * HB] + rec)
                    ybuf[c % 2, s, i * HB:(i + 1) * HB] = y
                    hs[i] = y.astype(jnp.bfloat16)
            out_cp(c).start()
        if NC >= 2:
            out_cp(NC - 2).wait()
        out_cp(NC - 1).wait()

    return body


def kernel(x_seq, h0, W, W_hidden):
    T, B, in_dim = x_seq.shape
    out_dim = W.shape[0]
    dtype = x_seq.dtype

    Dp = _ceil_to(in_dim, 128)
    Np = _ceil_to(out_dim, 128)
    TB = _ceil_to(B, 8)
    cs = _chunk_schedule(T)
    Cmax = max(cs)
    n_split = 2 if TB % 16 == 0 else 1
    HB = TB // n_split

    wt = jnp.zeros((Dp, Np), jnp.bfloat16).at[:in_dim, :out_dim].set(
        W.T.astype(jnp.bfloat16))
    wh = jnp.zeros((Dp, Np), jnp.bfloat16).at[:in_dim, :out_dim].set(
        W_hidden.T.astype(jnp.bfloat16))
    h0_p = jnp.zeros((TB, Dp), jnp.bfloat16).at[:B, :in_dim].set(
        h0.astype(jnp.bfloat16))
    if (B, in_dim) != (TB, Dp):
        x_p = jnp.zeros((T, TB, Dp), dtype).at[:, :B, :in_dim].set(x_seq)
    else:
        x_p = x_seq

    cost = pl.CostEstimate(
        flops=2 * 2 * T * TB * Dp * Np,
        transcendentals=T * TB * Np,
        bytes_accessed=4 * (2 * T * TB * Np),
    )

    out_p = pl.pallas_call(
        _make_body(cs, HB, n_split),
        out_shape=jax.ShapeDtypeStruct((T, TB, Np), jnp.float32),
        in_specs=[
            pl.BlockSpec(memory_space=pltpu.HBM),
            pl.BlockSpec(memory_space=pltpu.VMEM),
            pl.BlockSpec(memory_space=pltpu.VMEM),
            pl.BlockSpec(memory_space=pltpu.VMEM),
        ],
        out_specs=pl.BlockSpec(memory_space=pltpu.HBM),
        scratch_shapes=[
            pltpu.VMEM((3, Cmax, TB, Dp), jnp.float32),
            pltpu.VMEM((2, Cmax, TB, Np), jnp.float32),
            pltpu.SemaphoreType.DMA((3,)),
            pltpu.SemaphoreType.DMA((2,)),
        ],
        cost_estimate=cost,
    )(x_p, h0_p, wt, wh)

    if (TB, Np) != (B, out_dim):
        out_p = out_p[:, :B, :out_dim]
    return out_p.astype(dtype)
